# baseline (speedup 1.0000x reference)
import functools

import jax
import jax.numpy as jnp
from jax import lax
from jax.experimental import pallas as pl
from jax.experimental.pallas import tpu as pltpu
from jax.experimental.pallas import tpu_sc as plsc

NC = 2
NS = 16
NW = NC * NS
L = 16
CW = 128


def _make_tc_relayout(n_rows, n_cols):
    assert n_cols <= 2 * CW
    br = 200
    assert n_rows % br == 0

    def body(u_ref, t0_ref, t1_ref):
        x = u_ref[...][:, :, 0]
        t0_ref[...] = x[:, :CW]
        t1_ref[...] = x[:, n_cols - CW:n_cols]

    return pl.pallas_call(
        body,
        grid=(n_rows // br,),
        in_specs=[pl.BlockSpec((br, n_cols, 1), lambda i: (i, 0, 0))],
        out_specs=[pl.BlockSpec((br, CW), lambda i: (i, 0)),
                   pl.BlockSpec((br, CW), lambda i: (i, 0))],
        out_shape=[jax.ShapeDtypeStruct((n_rows, CW), jnp.float32),
                   jax.ShapeDtypeStruct((n_rows, CW), jnp.float32)],
    )


def _make_sc_kernel(n_points_q, n_frames_q):
    assert n_points_q % NW == 0
    b_per_w = n_points_q // NW
    chunk = 32
    n_chunks = b_per_w // chunk
    fgroups = (n_frames_q + L - 1) // L
    fpad = fgroups * L

    mesh = plsc.VectorSubcoreMesh(core_axis_name="c", subcore_axis_name="s")

    def body(offs_hbm, sel_hbm, pidx_hbm, t0_hbm, t1_hbm, out_hbm,
             offs_v, sel_v, pidx_v, rows_v, out_v, sem):
        c = lax.axis_index("c")
        s = lax.axis_index("s")
        wid = s * NC + c
        row0 = wid * b_per_w

        pltpu.sync_copy(offs_hbm, offs_v)
        pltpu.sync_copy(sel_hbm, sel_v)
        pltpu.sync_copy(pidx_hbm.at[pl.ds(row0, b_per_w)], pidx_v)
        f_off = [offs_v[pl.ds(g * L, L)] for g in range(fgroups)]
        f_sel = [sel_v[pl.ds(g * L, L)] for g in range(fgroups)]

        @pl.loop(0, n_chunks)
        def chunk_body(ch):
            idxs = pidx_v.at[pl.ds(ch * chunk, chunk)]
            d0 = pltpu.async_copy(t0_hbm.at[idxs],
                                  rows_v.at[pl.ds(0, chunk)], sem)
            d1 = pltpu.async_copy(t1_hbm.at[idxs],
                                  rows_v.at[pl.ds(chunk, chunk)], sem)
            d0.wait()
            d1.wait()

            def row_body(r, carry):
                rvec = jnp.full((L,), r, dtype=jnp.int32)
                for g in range(fgroups):
                    vals = plsc.load_gather(rows_v, [rvec + f_sel[g],
                                                     f_off[g]])
                    res = jnp.where(vals > 0.0, vals + 1.0, jnp.exp(vals))
                    out_v[pl.ds(r * n_frames_q + g * L, L)] = res
                return carry

            lax.fori_loop(0, chunk, row_body, 0)

            out_words = chunk * n_frames_q
            base = row0 + ch * chunk
            pltpu.sync_copy(out_v.at[pl.ds(0, out_words)],
                            out_hbm.at[pl.ds(base * n_frames_q, out_words)])

    kern = pl.kernel(
        body,
        out_type=jax.ShapeDtypeStruct((n_points_q * n_frames_q,), jnp.float32),
        mesh=mesh,
        scratch_types=[
            pltpu.VMEM((fpad,), jnp.int32),
            pltpu.VMEM((fpad,), jnp.int32),
            pltpu.VMEM((b_per_w,), jnp.int32),
            pltpu.VMEM((2 * chunk, CW), jnp.float32),
            pltpu.VMEM((chunk * n_frames_q + L,), jnp.float32),
            pltpu.SemaphoreType.DMA,
        ],
        compiler_params=pltpu.CompilerParams(needs_layout_passes=False),
    )
    return kern, fpad


def kernel(frames, points, uncertainty):
    n_rows, n_cols = uncertainty.shape[0], uncertainty.shape[1]
    p_q = points.shape[0]
    f_q = frames.shape[0]

    t0, t1 = _make_tc_relayout(n_rows, n_cols)(uncertainty)

    kern, fpad = _make_sc_kernel(p_q, f_q)
    f = frames.astype(jnp.int32)
    sel = jnp.where(f >= CW, 32, 0).astype(jnp.int32)
    offs = jnp.where(f >= CW, f - (n_cols - CW), f)
    pad = jnp.zeros((fpad - f_q,), dtype=jnp.int32)
    offs_pad = jnp.concatenate([offs, pad])
    sel_pad = jnp.concatenate([sel, pad])
    out = kern(offs_pad, sel_pad, points.astype(jnp.int32), t0, t1)
    return out.reshape(p_q, f_q, 1)


# speedup vs baseline: 32.0632x; 32.0632x over previous
import jax
import jax.numpy as jnp
from jax import lax
from jax.experimental import pallas as pl
from jax.experimental.pallas import tpu as pltpu
from jax.experimental.pallas import tpu_sc as plsc

NC = 2
NS = 16
NW = NC * NS
L = 16


def _make_sc_kernel(n_rows, n_points_q, n_frames_q):
    chunk = 2048
    assert n_points_q % chunk == 0
    n_chunks = n_points_q // chunk
    tasks = (n_frames_q + NW - 1) // NW

    mesh = plsc.VectorSubcoreMesh(core_axis_name="c", subcore_axis_name="s")

    def body(tab_hbm, pidx_hbm, out_hbm, row_v, pts_v, out_v, sem):
        c = lax.axis_index("c")
        s = lax.axis_index("s")
        wid = s * NC + c

        for t in range(tasks):
            j = wid + t * NW

            @pl.when(j < n_frames_q)
            def _():
                pltpu.sync_copy(tab_hbm.at[pl.ds(j * n_rows, n_rows)], row_v)

                @pl.loop(0, n_chunks)
                def chunk_body(ch):
                    pltpu.sync_copy(pidx_hbm.at[pl.ds(ch * chunk, chunk)],
                                    pts_v)

                    def grp_body(g, carry):
                        pvec = pts_v[pl.ds(g * L, L)]
                        vals = plsc.load_gather(row_v, [pvec])
                        res = jnp.where(vals > 0.0, vals + 1.0,
                                        jnp.exp(vals))
                        out_v[pl.ds(g * L, L)] = res
                        return carry

                    lax.fori_loop(0, chunk // L, grp_body, 0)
                    pltpu.sync_copy(
                        out_v,
                        out_hbm.at[pl.ds(j * n_points_q + ch * chunk, chunk)])

    kern = pl.kernel(
        body,
        out_type=jax.ShapeDtypeStruct((n_frames_q * n_points_q,), jnp.float32),
        mesh=mesh,
        scratch_types=[
            pltpu.VMEM((n_rows,), jnp.float32),
            pltpu.VMEM((chunk,), jnp.int32),
            pltpu.VMEM((chunk,), jnp.float32),
            pltpu.SemaphoreType.DMA,
        ],
        compiler_params=pltpu.CompilerParams(needs_layout_passes=False),
    )
    return kern


def kernel(frames, points, uncertainty):
    n_rows, n_cols = uncertainty.shape[0], uncertainty.shape[1]
    p_q = points.shape[0]
    f_q = frames.shape[0]

    ut = uncertainty.reshape(n_rows, n_cols).T
    tab = jnp.take(ut, frames.astype(jnp.int32), axis=0).reshape(-1)

    kern = _make_sc_kernel(n_rows, p_q, f_q)
    out_t = kern(tab, points.astype(jnp.int32))
    return out_t.reshape(f_q, p_q).T.reshape(p_q, f_q, 1)
